# baseline (speedup 1.0000x reference)
import jax
import jax.numpy as jnp
from jax import lax
from jax.experimental import pallas as pl
from jax.experimental.pallas import tpu as pltpu
from jax.experimental.pallas import tpu_sc as plsc

BATCH = 1024
VOCAB = 100000
BC = 2048
NJ = (VOCAB + BC - 1) // BC
K = int(0.7 * BATCH)


def _loss_body(x_ref, t_ref, loss_ref, m_acc, s_acc, t_acc):
    j = pl.program_id(0)

    @pl.when(j == 0)
    def _init():
        m_acc[...] = jnp.full((BATCH, 1), -jnp.inf, jnp.float32)
        s_acc[...] = jnp.zeros((BATCH, 1), jnp.float32)
        t_acc[...] = jnp.zeros((BATCH, 1), jnp.float32)

    x = x_ref[...]
    col = lax.broadcasted_iota(jnp.int32, (BATCH, BC), 1)
    limit = VOCAB - j * BC
    xm = jnp.where(col < limit, x, -jnp.inf)

    mj = jnp.max(xm, axis=1, keepdims=True)
    m_old = m_acc[...]
    m_new = jnp.maximum(m_old, mj)
    sj = jnp.sum(jnp.exp(xm - m_new), axis=1, keepdims=True)
    s_acc[...] = s_acc[...] * jnp.exp(m_old - m_new) + sj
    m_acc[...] = m_new

    tshift = t_ref[...] - j * BC
    t_acc[...] = t_acc[...] + jnp.sum(
        jnp.where(col == tshift, xm, 0.0), axis=1, keepdims=True)

    @pl.when(j == NJ - 1)
    def _fin():
        loss_ref[...] = m_acc[...] + jnp.log(s_acc[...]) - t_acc[...]


def _per_sample_loss(x, tgt2d):
    return pl.pallas_call(
        _loss_body,
        grid=(NJ,),
        in_specs=[
            pl.BlockSpec((BATCH, BC), lambda j: (0, j)),
            pl.BlockSpec((BATCH, 1), lambda j: (0, 0)),
        ],
        out_specs=pl.BlockSpec((BATCH, 1), lambda j: (0, 0)),
        out_shape=jax.ShapeDtypeStruct((BATCH, 1), jnp.float32),
        scratch_shapes=[
            pltpu.VMEM((BATCH, 1), jnp.float32),
            pltpu.VMEM((BATCH, 1), jnp.float32),
            pltpu.VMEM((BATCH, 1), jnp.float32),
        ],
        compiler_params=pltpu.CompilerParams(
            dimension_semantics=("arbitrary",)),
    )(x, tgt2d)


_NVR = BATCH // 16
_I32_MIN = -2147483648
_I32_FLIP = 0x7FFFFFFF


def _topk_body(loss_hbm, out_hbm, loss_v, key_v, out_v):
    c = lax.axis_index("c")
    s = lax.axis_index("s")

    @pl.when(jnp.logical_and(c == 0, s == 0))
    def _work():
        pltpu.sync_copy(loss_hbm, loss_v)

        for i in range(_NVR):
            b = plsc.bitcast(loss_v[pl.ds(i * 16, 16)], jnp.int32)
            key_v[pl.ds(i * 16, 16)] = jnp.where(b < 0, b ^ _I32_FLIP, b)

        def count_ge(cand):
            acc = jnp.zeros((16,), jnp.int32)
            for i in range(_NVR):
                kv = key_v[pl.ds(i * 16, 16)]
                acc = acc + jnp.where(kv >= cand, 1, 0).astype(jnp.int32)
            return jnp.sum(acc)

        t0 = jnp.where(count_ge(jnp.int32(0)) >= K,
                       jnp.int32(0), jnp.int32(_I32_MIN))

        def bit_step(i, t):
            cand = t | lax.shift_left(jnp.int32(1), jnp.int32(30) - i)
            return jnp.where(count_ge(cand) >= K, cand, t)

        t = lax.fori_loop(0, 31, bit_step, t0)

        acc_sum = jnp.zeros((16,), jnp.float32)
        acc_cnt = jnp.zeros((16,), jnp.int32)
        for i in range(_NVR):
            kv = key_v[pl.ds(i * 16, 16)]
            xv = loss_v[pl.ds(i * 16, 16)]
            m = kv > t
            acc_sum = acc_sum + jnp.where(m, xv, 0.0)
            acc_cnt = acc_cnt + jnp.where(m, 1, 0).astype(jnp.int32)
        sum_gt = jnp.sum(acc_sum)
        cnt_gt = jnp.sum(acc_cnt)

        tbits = jnp.where(t < 0, t ^ _I32_FLIP, t)
        tval = plsc.bitcast(jnp.full((16,), tbits, jnp.int32), jnp.float32)
        mean_vec = (sum_gt + (K - cnt_gt).astype(jnp.float32) * tval) * (1.0 / K)
        out_v[...] = mean_vec
        pltpu.sync_copy(out_v, out_hbm)


def _topk_mean(loss1d):
    fn = pl.kernel(
        _topk_body,
        out_type=jax.ShapeDtypeStruct((16,), jnp.float32),
        mesh=plsc.VectorSubcoreMesh(core_axis_name="c", subcore_axis_name="s"),
        scratch_types=[
            pltpu.VMEM((BATCH,), jnp.float32),
            pltpu.VMEM((BATCH,), jnp.int32),
            pltpu.VMEM((16,), jnp.float32),
        ],
        compiler_params=pltpu.CompilerParams(needs_layout_passes=False),
    )
    return fn(loss1d)


def kernel(x, target):
    tgt2d = target.astype(jnp.int32).reshape(BATCH, 1)
    loss = _per_sample_loss(x, tgt2d)
    out16 = _topk_mean(loss.reshape(BATCH))
    return out16[0]
